# baseline (speedup 1.0000x reference)
import functools

import jax
import jax.numpy as jnp
from jax import lax
from jax.experimental import pallas as pl
from jax.experimental.pallas import tpu as pltpu
from jax.experimental.pallas import tpu_sc as plsc

H = 768
F = 512
E = 64
K = 2
C = 192
T = 2048
A = T * K
CHUNK = 128
NCH = A // CHUNK


def _routing_body(ids_ref, w_ref, rowd_ref, rowc_ref, scale_ref, tok_ref,
                  oh_ref, cum_ref):
    ids = ids_ref[...]
    eidx = lax.broadcasted_iota(jnp.int32, (1, E), 1)
    oh_ref[...] = (ids == eidx).astype(jnp.float32)
    tri = (lax.broadcasted_iota(jnp.int32, (CHUNK, CHUNK), 0)
           >= lax.broadcasted_iota(jnp.int32, (CHUNK, CHUNK), 1)
           ).astype(jnp.float32)

    def step(i, carry):
        oh_c = oh_ref[pl.ds(i * CHUNK, CHUNK), :]
        cum = lax.dot_general(tri, oh_c, (((1,), (0,)), ((), ())),
                              preferred_element_type=jnp.float32) + carry
        cum_ref[pl.ds(i * CHUNK, CHUNK), :] = cum
        return lax.slice(cum, (CHUNK - 1, 0), (CHUNK, E))

    lax.fori_loop(0, NCH, step, jnp.zeros((1, E), jnp.float32))

    pos = (jnp.sum(cum_ref[...] * oh_ref[...], axis=1, keepdims=True)
           .astype(jnp.int32) - 1)
    valid = pos < C
    slot = jnp.where(valid, pos, 0)
    rowc_ref[...] = ids * C + slot
    rowd_ref[...] = jnp.where(valid, ids * C + pos, E * C)
    scale = jnp.where(valid, w_ref[...], 0.0)
    scale_ref[...] = jnp.broadcast_to(scale, (A, 16))
    tok_ref[...] = lax.broadcasted_iota(jnp.int32, (A, 1), 0) // K


def _routing(ids_flat, w_flat):
    i32 = jnp.int32
    return pl.pallas_call(
        _routing_body,
        out_shape=[
            jax.ShapeDtypeStruct((A, 1), i32),
            jax.ShapeDtypeStruct((A, 1), i32),
            jax.ShapeDtypeStruct((A, 16), jnp.float32),
            jax.ShapeDtypeStruct((A, 1), i32),
        ],
        scratch_shapes=[
            pltpu.VMEM((A, E), jnp.float32),
            pltpu.VMEM((A, E), jnp.float32),
        ],
    )(ids_flat, w_flat)


def _sc_dispatch(hidden, tok, rowd):
    info = plsc.get_sparse_core_info()
    nc = info.num_cores
    mesh = plsc.VectorSubcoreMesh(core_axis_name="c", subcore_axis_name="s")

    @functools.partial(
        pl.kernel, mesh=mesh,
        out_type=jax.ShapeDtypeStruct(((E + 1) * C, H // 2), jnp.int32),
        scratch_types=[
            pltpu.VMEM((CHUNK,), jnp.int32),
            pltpu.VMEM((CHUNK,), jnp.int32),
            pltpu.VMEM((CHUNK, H // 2), jnp.int32),
            pltpu.SemaphoreType.DMA,
        ],
    )
    def k(hid_hbm, tok_hbm, rowd_hbm, out_hbm, tok_v, row_v, rows_v, sem):
        wid = lax.axis_index("s") * nc + lax.axis_index("c")
        base = wid * CHUNK
        pltpu.sync_copy(tok_hbm.at[pl.ds(base, CHUNK)], tok_v)
        pltpu.sync_copy(rowd_hbm.at[pl.ds(base, CHUNK)], row_v)
        pltpu.async_copy(hid_hbm.at[tok_v], rows_v, sem).wait()
        pltpu.async_copy(rows_v, out_hbm.at[row_v], sem).wait()

    return k(hidden, tok, rowd)


def _ffn_body(x_ref, gw_ref, dw_ref, out_ref):
    x = x_ref[0]
    gw = gw_ref[0].astype(jnp.bfloat16)
    gu = lax.dot_general(x, gw, (((1,), (1,)), ((), ())),
                         preferred_element_type=jnp.float32)
    gate = gu[:, :F]
    up = gu[:, F:]
    act = (gate * jax.nn.sigmoid(gate) * up).astype(jnp.bfloat16)
    dw = dw_ref[0].astype(jnp.bfloat16)
    out_ref[0] = lax.dot_general(act, dw, (((1,), (1,)), ((), ())),
                                 preferred_element_type=jnp.float32)


def _ffn(expert_in, gate_up_weight, down_weight):
    return pl.pallas_call(
        _ffn_body,
        grid=(E,),
        in_specs=[
            pl.BlockSpec((1, C, H), lambda e: (e, 0, 0)),
            pl.BlockSpec((1, 2 * F, H), lambda e: (e, 0, 0)),
            pl.BlockSpec((1, H, F), lambda e: (e, 0, 0)),
        ],
        out_specs=pl.BlockSpec((1, C, H), lambda e: (e, 0, 0)),
        out_shape=jax.ShapeDtypeStruct((E, C, H), jnp.float32),
    )(expert_in, gate_up_weight, down_weight)


HW = 64


def _sc_combine(eo_flat, rowc, scale_b):
    info = plsc.get_sparse_core_info()
    nc = info.num_cores
    mesh = plsc.VectorSubcoreMesh(core_axis_name="c", subcore_axis_name="s")

    @functools.partial(
        pl.kernel, mesh=mesh,
        out_type=jax.ShapeDtypeStruct((T, H), jnp.float32),
        scratch_types=[
            pltpu.VMEM((HW,), jnp.int32),
            pltpu.VMEM((HW, 16), jnp.float32),
            pltpu.VMEM((HW, H), jnp.float32),
            pltpu.VMEM((HW // K, H), jnp.float32),
            pltpu.SemaphoreType.DMA,
        ],
    )
    def k(eo_hbm, rowc_hbm, sb_hbm, out_hbm, row_v, sb_v, rows_v, out_v, sem):
        wid = lax.axis_index("s") * nc + lax.axis_index("c")
        for h in range(CHUNK // HW):
            ab = pl.multiple_of(wid * CHUNK + h * HW, HW)
            tb = pl.multiple_of((wid * CHUNK + h * HW) // K, HW // K)
            pltpu.sync_copy(rowc_hbm.at[pl.ds(ab, HW)], row_v)
            pltpu.sync_copy(sb_hbm.at[pl.ds(ab, HW)], sb_v)
            pltpu.async_copy(eo_hbm.at[row_v], rows_v, sem).wait()

            def body(i, carry):
                s0 = sb_v[2 * i, :]
                s1 = sb_v[2 * i + 1, :]
                for j in range(H // 16):
                    r0 = rows_v[2 * i, pl.ds(j * 16, 16)]
                    r1 = rows_v[2 * i + 1, pl.ds(j * 16, 16)]
                    out_v[i, pl.ds(j * 16, 16)] = r0 * s0 + r1 * s1
                return carry

            lax.fori_loop(0, HW // K, body, 0)
            pltpu.sync_copy(out_v, out_hbm.at[pl.ds(tb, HW // K)])

    return k(eo_flat, rowc, scale_b)


@jax.jit
def kernel(hidden_states, topk_weights, topk_ids, gate_up_weight, down_weight):
    ids_flat = topk_ids.reshape(A, 1).astype(jnp.int32)
    w_flat = topk_weights.reshape(A, 1)
    rowd, rowc, scale, tok = _routing(ids_flat, w_flat)
    hid32 = lax.bitcast_convert_type(
        hidden_states.astype(jnp.bfloat16).reshape(T, H // 2, 2), jnp.int32)
    ein32 = _sc_dispatch(hid32, tok.reshape(A), rowd.reshape(A))
    ein = lax.bitcast_convert_type(ein32, jnp.bfloat16).reshape(E + 1, C, H)
    eo = _ffn(ein, gate_up_weight, down_weight)
    return _sc_combine(eo.reshape(E * C, H), rowc.reshape(A), scale)


# speedup vs baseline: 1.9173x; 1.9173x over previous
import functools

import jax
import jax.numpy as jnp
from jax import lax
from jax.experimental import pallas as pl
from jax.experimental.pallas import tpu as pltpu
from jax.experimental.pallas import tpu_sc as plsc

H = 768
F = 512
E = 64
K = 2
C = 192
T = 2048
A = T * K
CHUNK = 128
NCH = A // CHUNK


def _routing_body(ids_ref, w_ref, rowd_ref, rowc_ref, scale_ref, tok_ref,
                  oh_ref, cum_ref):
    ids = ids_ref[...]
    eidx = lax.broadcasted_iota(jnp.int32, (1, E), 1)
    oh_ref[...] = (ids == eidx).astype(jnp.float32)
    tri = (lax.broadcasted_iota(jnp.int32, (CHUNK, CHUNK), 0)
           >= lax.broadcasted_iota(jnp.int32, (CHUNK, CHUNK), 1)
           ).astype(jnp.float32)

    def step(i, carry):
        oh_c = oh_ref[pl.ds(i * CHUNK, CHUNK), :]
        cum = lax.dot_general(tri, oh_c, (((1,), (0,)), ((), ())),
                              preferred_element_type=jnp.float32) + carry
        cum_ref[pl.ds(i * CHUNK, CHUNK), :] = cum
        return lax.slice(cum, (CHUNK - 1, 0), (CHUNK, E))

    lax.fori_loop(0, NCH, step, jnp.zeros((1, E), jnp.float32))

    pos = (jnp.sum(cum_ref[...] * oh_ref[...], axis=1, keepdims=True)
           .astype(jnp.int32) - 1)
    valid = pos < C
    slot = jnp.where(valid, pos, 0)
    rowc_ref[...] = ids * C + slot
    rowd_ref[...] = jnp.where(valid, ids * C + pos, E * C)
    scale = jnp.where(valid, w_ref[...], 0.0)
    scale_ref[...] = jnp.broadcast_to(scale, (A, 16))
    tok_ref[...] = lax.broadcasted_iota(jnp.int32, (A, 1), 0) // K


def _routing(ids_flat, w_flat):
    i32 = jnp.int32
    return pl.pallas_call(
        _routing_body,
        out_shape=[
            jax.ShapeDtypeStruct((A, 1), i32),
            jax.ShapeDtypeStruct((A, 1), i32),
            jax.ShapeDtypeStruct((A, 16), jnp.float32),
            jax.ShapeDtypeStruct((A, 1), i32),
        ],
        scratch_shapes=[
            pltpu.VMEM((A, E), jnp.float32),
            pltpu.VMEM((A, E), jnp.float32),
        ],
    )(ids_flat, w_flat)


def _sc_dispatch(hidden, tok, rowd):
    info = plsc.get_sparse_core_info()
    nc = info.num_cores
    mesh = plsc.VectorSubcoreMesh(core_axis_name="c", subcore_axis_name="s")

    @functools.partial(
        pl.kernel, mesh=mesh,
        out_type=jax.ShapeDtypeStruct(((E + 1) * C, H), jnp.float32),
        scratch_types=[
            pltpu.VMEM((CHUNK,), jnp.int32),
            pltpu.VMEM((CHUNK,), jnp.int32),
            pltpu.VMEM((CHUNK, H), jnp.float32),
            pltpu.SemaphoreType.DMA,
        ],
    )
    def k(hid_hbm, tok_hbm, rowd_hbm, out_hbm, tok_v, row_v, rows_v, sem):
        wid = lax.axis_index("s") * nc + lax.axis_index("c")
        base = wid * CHUNK
        pltpu.sync_copy(tok_hbm.at[pl.ds(base, CHUNK)], tok_v)
        pltpu.sync_copy(rowd_hbm.at[pl.ds(base, CHUNK)], row_v)
        pltpu.async_copy(hid_hbm.at[tok_v], rows_v, sem).wait()
        pltpu.async_copy(rows_v, out_hbm.at[row_v], sem).wait()

    return k(hidden, tok, rowd)


def _ffn_body(x_ref, gw_ref, dw_ref, out_ref):
    x = x_ref[0].astype(jnp.bfloat16)
    gw = gw_ref[0].astype(jnp.bfloat16)
    gu = lax.dot_general(x, gw, (((1,), (1,)), ((), ())),
                         preferred_element_type=jnp.float32)
    gate = gu[:, :F]
    up = gu[:, F:]
    act = (gate * jax.nn.sigmoid(gate) * up).astype(jnp.bfloat16)
    dw = dw_ref[0].astype(jnp.bfloat16)
    out_ref[0] = lax.dot_general(act, dw, (((1,), (1,)), ((), ())),
                                 preferred_element_type=jnp.float32)


def _ffn(expert_in, gate_up_weight, down_weight):
    return pl.pallas_call(
        _ffn_body,
        grid=(E,),
        in_specs=[
            pl.BlockSpec((1, C, H), lambda e: (e, 0, 0)),
            pl.BlockSpec((1, 2 * F, H), lambda e: (e, 0, 0)),
            pl.BlockSpec((1, H, F), lambda e: (e, 0, 0)),
        ],
        out_specs=pl.BlockSpec((1, C, H), lambda e: (e, 0, 0)),
        out_shape=jax.ShapeDtypeStruct((E, C, H), jnp.float32),
    )(expert_in, gate_up_weight, down_weight)


HW = 64


def _sc_combine(eo_flat, rowc, scale_b):
    info = plsc.get_sparse_core_info()
    nc = info.num_cores
    mesh = plsc.VectorSubcoreMesh(core_axis_name="c", subcore_axis_name="s")

    @functools.partial(
        pl.kernel, mesh=mesh,
        out_type=jax.ShapeDtypeStruct((T, H), jnp.float32),
        scratch_types=[
            pltpu.VMEM((HW,), jnp.int32),
            pltpu.VMEM((HW, 16), jnp.float32),
            pltpu.VMEM((HW, H), jnp.float32),
            pltpu.VMEM((HW // K, H), jnp.float32),
            pltpu.SemaphoreType.DMA,
        ],
    )
    def k(eo_hbm, rowc_hbm, sb_hbm, out_hbm, row_v, sb_v, rows_v, out_v, sem):
        wid = lax.axis_index("s") * nc + lax.axis_index("c")
        for h in range(CHUNK // HW):
            ab = pl.multiple_of(wid * CHUNK + h * HW, HW)
            tb = pl.multiple_of((wid * CHUNK + h * HW) // K, HW // K)
            pltpu.sync_copy(rowc_hbm.at[pl.ds(ab, HW)], row_v)
            pltpu.sync_copy(sb_hbm.at[pl.ds(ab, HW)], sb_v)
            pltpu.async_copy(eo_hbm.at[row_v], rows_v, sem).wait()

            def body(i, carry):
                s0 = sb_v[2 * i, :]
                s1 = sb_v[2 * i + 1, :]
                for j in range(H // 16):
                    r0 = rows_v[2 * i, pl.ds(j * 16, 16)]
                    r1 = rows_v[2 * i + 1, pl.ds(j * 16, 16)]
                    out_v[i, pl.ds(j * 16, 16)] = r0 * s0 + r1 * s1
                return carry

            lax.fori_loop(0, HW // K, body, 0)
            pltpu.sync_copy(out_v, out_hbm.at[pl.ds(tb, HW // K)])

    return k(eo_flat, rowc, scale_b)


@jax.jit
def kernel(hidden_states, topk_weights, topk_ids, gate_up_weight, down_weight):
    ids_flat = topk_ids.reshape(A, 1).astype(jnp.int32)
    w_flat = topk_weights.reshape(A, 1)
    rowd, rowc, scale, tok = _routing(ids_flat, w_flat)
    expert_in = _sc_dispatch(hidden_states, tok.reshape(A), rowd.reshape(A))
    eo = _ffn(expert_in.reshape(E + 1, C, H), gate_up_weight, down_weight)
    return _sc_combine(eo.reshape(E * C, H), rowc.reshape(A), scale)


# speedup vs baseline: 1.9745x; 1.0298x over previous
import functools

import jax
import jax.numpy as jnp
from jax import lax
from jax.experimental import pallas as pl
from jax.experimental.pallas import tpu as pltpu
from jax.experimental.pallas import tpu_sc as plsc

H = 768
F = 512
E = 64
K = 2
C = 192
T = 2048
A = T * K
CHUNK = 128
NCH = A // CHUNK


def _routing_body(ids_ref, w_ref, hid_ref, rowd_ref, rowc_ref, scale_ref,
                  tok_ref, hb_ref, oh_ref, cum_ref):
    xb = hid_ref[...].astype(jnp.bfloat16)
    lo = lax.bitcast_convert_type(xb[:, :H // 2], jnp.uint16).astype(jnp.uint32)
    hi = lax.bitcast_convert_type(xb[:, H // 2:], jnp.uint16).astype(jnp.uint32)
    hb_ref[...] = lax.bitcast_convert_type(lo | (hi << 16), jnp.int32)

    ids = ids_ref[...]
    eidx = lax.broadcasted_iota(jnp.int32, (1, E), 1)
    oh_ref[...] = (ids == eidx).astype(jnp.float32)
    tri = (lax.broadcasted_iota(jnp.int32, (CHUNK, CHUNK), 0)
           >= lax.broadcasted_iota(jnp.int32, (CHUNK, CHUNK), 1)
           ).astype(jnp.float32)

    def step(i, carry):
        oh_c = oh_ref[pl.ds(i * CHUNK, CHUNK), :]
        cum = lax.dot_general(tri, oh_c, (((1,), (0,)), ((), ())),
                              preferred_element_type=jnp.float32) + carry
        cum_ref[pl.ds(i * CHUNK, CHUNK), :] = cum
        return lax.slice(cum, (CHUNK - 1, 0), (CHUNK, E))

    lax.fori_loop(0, NCH, step, jnp.zeros((1, E), jnp.float32))

    pos = (jnp.sum(cum_ref[...] * oh_ref[...], axis=1, keepdims=True)
           .astype(jnp.int32) - 1)
    valid = pos < C
    slot = jnp.where(valid, pos, 0)
    rowc_ref[...] = ids * C + slot
    rowd_ref[...] = jnp.where(valid, ids * C + pos, E * C)
    scale = jnp.where(valid, w_ref[...], 0.0)
    scale_ref[...] = jnp.broadcast_to(scale, (A, 16))
    tok_ref[...] = lax.broadcasted_iota(jnp.int32, (A, 1), 0) // K


def _routing(ids_flat, w_flat, hidden):
    i32 = jnp.int32
    return pl.pallas_call(
        _routing_body,
        out_shape=[
            jax.ShapeDtypeStruct((A, 1), i32),
            jax.ShapeDtypeStruct((A, 1), i32),
            jax.ShapeDtypeStruct((A, 16), jnp.float32),
            jax.ShapeDtypeStruct((A, 1), i32),
            jax.ShapeDtypeStruct((T, H // 2), i32),
        ],
        scratch_shapes=[
            pltpu.VMEM((A, E), jnp.float32),
            pltpu.VMEM((A, E), jnp.float32),
        ],
    )(ids_flat, w_flat, hidden)


def _sc_dispatch(hidden, tok, rowd):
    info = plsc.get_sparse_core_info()
    nc = info.num_cores
    mesh = plsc.VectorSubcoreMesh(core_axis_name="c", subcore_axis_name="s")

    @functools.partial(
        pl.kernel, mesh=mesh,
        out_type=jax.ShapeDtypeStruct(((E + 1) * C, H // 2), jnp.int32),
        scratch_types=[
            pltpu.VMEM((CHUNK,), jnp.int32),
            pltpu.VMEM((CHUNK,), jnp.int32),
            pltpu.VMEM((CHUNK, H // 2), jnp.int32),
            pltpu.SemaphoreType.DMA,
        ],
    )
    def k(hid_hbm, tok_hbm, rowd_hbm, out_hbm, tok_v, row_v, rows_v, sem):
        wid = lax.axis_index("s") * nc + lax.axis_index("c")
        base = wid * CHUNK
        pltpu.sync_copy(tok_hbm.at[pl.ds(base, CHUNK)], tok_v)
        pltpu.sync_copy(rowd_hbm.at[pl.ds(base, CHUNK)], row_v)
        pltpu.async_copy(hid_hbm.at[tok_v], rows_v, sem).wait()
        pltpu.async_copy(rows_v, out_hbm.at[row_v], sem).wait()

    return k(hidden, tok, rowd)


def _ffn_body(x_ref, gw_ref, dw_ref, out_ref):
    xu = lax.bitcast_convert_type(x_ref[0], jnp.uint32)
    x_lo = lax.bitcast_convert_type((xu & 0xffff).astype(jnp.uint16),
                                    jnp.bfloat16)
    x_hi = lax.bitcast_convert_type((xu >> 16).astype(jnp.uint16),
                                    jnp.bfloat16)
    x = jnp.concatenate([x_lo, x_hi], axis=1)
    gw = gw_ref[0].astype(jnp.bfloat16)
    gu = lax.dot_general(x, gw, (((1,), (1,)), ((), ())),
                         preferred_element_type=jnp.float32)
    gate = gu[:, :F]
    up = gu[:, F:]
    act = (gate * jax.nn.sigmoid(gate) * up).astype(jnp.bfloat16)
    dw = dw_ref[0].astype(jnp.bfloat16)
    out_ref[0] = lax.dot_general(act, dw, (((1,), (1,)), ((), ())),
                                 preferred_element_type=jnp.float32)


def _ffn(expert_in, gate_up_weight, down_weight):
    return pl.pallas_call(
        _ffn_body,
        grid=(E,),
        in_specs=[
            pl.BlockSpec((1, C, H // 2), lambda e: (e, 0, 0)),
            pl.BlockSpec((1, 2 * F, H), lambda e: (e, 0, 0)),
            pl.BlockSpec((1, H, F), lambda e: (e, 0, 0)),
        ],
        out_specs=pl.BlockSpec((1, C, H), lambda e: (e, 0, 0)),
        out_shape=jax.ShapeDtypeStruct((E, C, H), jnp.float32),
    )(expert_in, gate_up_weight, down_weight)


HW = 64


def _sc_combine(eo_flat, rowc, scale_b):
    info = plsc.get_sparse_core_info()
    nc = info.num_cores
    mesh = plsc.VectorSubcoreMesh(core_axis_name="c", subcore_axis_name="s")

    @functools.partial(
        pl.kernel, mesh=mesh,
        out_type=jax.ShapeDtypeStruct((T, H), jnp.float32),
        scratch_types=[
            pltpu.VMEM((HW,), jnp.int32),
            pltpu.VMEM((HW, 16), jnp.float32),
            pltpu.VMEM((HW, H), jnp.float32),
            pltpu.VMEM((HW // K, H), jnp.float32),
            pltpu.SemaphoreType.DMA,
        ],
    )
    def k(eo_hbm, rowc_hbm, sb_hbm, out_hbm, row_v, sb_v, rows_v, out_v, sem):
        wid = lax.axis_index("s") * nc + lax.axis_index("c")
        for h in range(CHUNK // HW):
            ab = pl.multiple_of(wid * CHUNK + h * HW, HW)
            tb = pl.multiple_of((wid * CHUNK + h * HW) // K, HW // K)
            pltpu.sync_copy(rowc_hbm.at[pl.ds(ab, HW)], row_v)
            pltpu.sync_copy(sb_hbm.at[pl.ds(ab, HW)], sb_v)
            pltpu.async_copy(eo_hbm.at[row_v], rows_v, sem).wait()

            def body(i, carry):
                s0 = sb_v[2 * i, :]
                s1 = sb_v[2 * i + 1, :]
                for j in range(H // 16):
                    r0 = rows_v[2 * i, pl.ds(j * 16, 16)]
                    r1 = rows_v[2 * i + 1, pl.ds(j * 16, 16)]
                    out_v[i, pl.ds(j * 16, 16)] = r0 * s0 + r1 * s1
                return carry

            lax.fori_loop(0, HW // K, body, 0)
            pltpu.sync_copy(out_v, out_hbm.at[pl.ds(tb, HW // K)])

    return k(eo_flat, rowc, scale_b)


@jax.jit
def kernel(hidden_states, topk_weights, topk_ids, gate_up_weight, down_weight):
    ids_flat = topk_ids.reshape(A, 1).astype(jnp.int32)
    w_flat = topk_weights.reshape(A, 1)
    rowd, rowc, scale, tok, hb32 = _routing(ids_flat, w_flat, hidden_states)
    expert_in = _sc_dispatch(hb32, tok.reshape(A), rowd.reshape(A))
    eo = _ffn(expert_in.reshape(E + 1, C, H // 2), gate_up_weight, down_weight)
    return _sc_combine(eo.reshape(E * C, H), rowc.reshape(A), scale)


# speedup vs baseline: 2.1122x; 1.0697x over previous
import functools

import jax
import jax.numpy as jnp
from jax import lax
from jax.experimental import pallas as pl
from jax.experimental.pallas import tpu as pltpu
from jax.experimental.pallas import tpu_sc as plsc

H = 768
F = 512
E = 64
K = 2
C = 192
T = 2048
A = T * K
CHUNK = 128
NCH = A // CHUNK


def _routing_body(ids_ref, w_ref, hid_ref, rowd_ref, rowc_ref, scale_ref,
                  tok_ref, hb_ref, oh_ref, cum_ref):
    xb = hid_ref[...].astype(jnp.bfloat16)
    lo = lax.bitcast_convert_type(xb[:, :H // 2], jnp.uint16).astype(jnp.uint32)
    hi = lax.bitcast_convert_type(xb[:, H // 2:], jnp.uint16).astype(jnp.uint32)
    hb_ref[...] = lax.bitcast_convert_type(lo | (hi << 16), jnp.int32)

    ids = ids_ref[...]
    eidx = lax.broadcasted_iota(jnp.int32, (1, E), 1)
    oh_ref[...] = (ids == eidx).astype(jnp.float32)
    tri = (lax.broadcasted_iota(jnp.int32, (CHUNK, CHUNK), 0)
           >= lax.broadcasted_iota(jnp.int32, (CHUNK, CHUNK), 1)
           ).astype(jnp.float32)

    def step(i, carry):
        oh_c = oh_ref[pl.ds(i * CHUNK, CHUNK), :]
        cum = lax.dot_general(tri, oh_c, (((1,), (0,)), ((), ())),
                              preferred_element_type=jnp.float32) + carry
        cum_ref[pl.ds(i * CHUNK, CHUNK), :] = cum
        return lax.slice(cum, (CHUNK - 1, 0), (CHUNK, E))

    lax.fori_loop(0, NCH, step, jnp.zeros((1, E), jnp.float32))

    pos = (jnp.sum(cum_ref[...] * oh_ref[...], axis=1, keepdims=True)
           .astype(jnp.int32) - 1)
    valid = pos < C
    slot = jnp.where(valid, pos, 0)
    rowc_ref[...] = ids * C + slot
    rowd_ref[...] = jnp.where(valid, ids * C + pos, E * C)
    scale = jnp.where(valid, w_ref[...], 0.0)
    scale_ref[...] = jnp.broadcast_to(scale, (A, 16))
    tok_ref[...] = lax.broadcasted_iota(jnp.int32, (A, 1), 0) // K


def _routing(ids_flat, w_flat, hidden):
    i32 = jnp.int32
    return pl.pallas_call(
        _routing_body,
        out_shape=[
            jax.ShapeDtypeStruct((A, 1), i32),
            jax.ShapeDtypeStruct((A, 1), i32),
            jax.ShapeDtypeStruct((A, 16), jnp.float32),
            jax.ShapeDtypeStruct((A, 1), i32),
            jax.ShapeDtypeStruct((T, H // 2), i32),
        ],
        scratch_shapes=[
            pltpu.VMEM((A, E), jnp.float32),
            pltpu.VMEM((A, E), jnp.float32),
        ],
    )(ids_flat, w_flat, hidden)


def _sc_dispatch(hidden, tok, rowd):
    info = plsc.get_sparse_core_info()
    nc = info.num_cores
    mesh = plsc.VectorSubcoreMesh(core_axis_name="c", subcore_axis_name="s")

    @functools.partial(
        pl.kernel, mesh=mesh,
        out_type=jax.ShapeDtypeStruct(((E + 1) * C, H // 2), jnp.int32),
        scratch_types=[
            pltpu.VMEM((CHUNK,), jnp.int32),
            pltpu.VMEM((CHUNK,), jnp.int32),
            pltpu.VMEM((CHUNK, H // 2), jnp.int32),
            pltpu.SemaphoreType.DMA,
        ],
    )
    def k(hid_hbm, tok_hbm, rowd_hbm, out_hbm, tok_v, row_v, rows_v, sem):
        wid = lax.axis_index("s") * nc + lax.axis_index("c")
        base = wid * CHUNK
        pltpu.sync_copy(tok_hbm.at[pl.ds(base, CHUNK)], tok_v)
        pltpu.sync_copy(rowd_hbm.at[pl.ds(base, CHUNK)], row_v)
        pltpu.async_copy(hid_hbm.at[tok_v], rows_v, sem).wait()
        pltpu.async_copy(rows_v, out_hbm.at[row_v], sem).wait()

    return k(hidden, tok, rowd)


def _ffn_body(x_ref, gw_ref, dw_ref, out_ref):
    xu = lax.bitcast_convert_type(x_ref[0], jnp.uint32)
    x_lo = lax.bitcast_convert_type((xu & 0xffff).astype(jnp.uint16),
                                    jnp.bfloat16)
    x_hi = lax.bitcast_convert_type((xu >> 16).astype(jnp.uint16),
                                    jnp.bfloat16)
    x = jnp.concatenate([x_lo, x_hi], axis=1)
    gw = gw_ref[0].astype(jnp.bfloat16)
    gu = lax.dot_general(x, gw, (((1,), (1,)), ((), ())),
                         preferred_element_type=jnp.float32)
    gate = gu[:, :F]
    up = gu[:, F:]
    act = (gate * jax.nn.sigmoid(gate) * up).astype(jnp.bfloat16)
    dw = dw_ref[0].astype(jnp.bfloat16)
    eo = lax.dot_general(act, dw, (((1,), (1,)), ((), ())),
                         preferred_element_type=jnp.float32)
    eb = eo.astype(jnp.bfloat16)
    lo = lax.bitcast_convert_type(eb[:, :H // 2], jnp.uint16).astype(jnp.uint32)
    hi = lax.bitcast_convert_type(eb[:, H // 2:], jnp.uint16).astype(jnp.uint32)
    out_ref[0] = lax.bitcast_convert_type(lo | (hi << 16), jnp.int32)


def _ffn(expert_in, gate_up_weight, down_weight):
    return pl.pallas_call(
        _ffn_body,
        grid=(E,),
        in_specs=[
            pl.BlockSpec((1, C, H // 2), lambda e: (e, 0, 0)),
            pl.BlockSpec((1, 2 * F, H), lambda e: (e, 0, 0)),
            pl.BlockSpec((1, H, F), lambda e: (e, 0, 0)),
        ],
        out_specs=pl.BlockSpec((1, C, H // 2), lambda e: (e, 0, 0)),
        out_shape=jax.ShapeDtypeStruct((E, C, H // 2), jnp.int32),
    )(expert_in, gate_up_weight, down_weight)


HW = 64


def _sc_combine(eo_flat, rowc, scale_b):
    info = plsc.get_sparse_core_info()
    nc = info.num_cores
    mesh = plsc.VectorSubcoreMesh(core_axis_name="c", subcore_axis_name="s")

    @functools.partial(
        pl.kernel, mesh=mesh,
        out_type=jax.ShapeDtypeStruct((T, H), jnp.float32),
        scratch_types=[
            pltpu.VMEM((HW,), jnp.int32),
            pltpu.VMEM((HW, 16), jnp.float32),
            pltpu.VMEM((HW, H // 2), jnp.int32),
            pltpu.VMEM((HW // K, H), jnp.float32),
            pltpu.SemaphoreType.DMA,
        ],
    )
    def k(eo_hbm, rowc_hbm, sb_hbm, out_hbm, row_v, sb_v, rows_v, out_v, sem):
        wid = lax.axis_index("s") * nc + lax.axis_index("c")
        for h in range(CHUNK // HW):
            ab = pl.multiple_of(wid * CHUNK + h * HW, HW)
            tb = pl.multiple_of((wid * CHUNK + h * HW) // K, HW // K)
            pltpu.sync_copy(rowc_hbm.at[pl.ds(ab, HW)], row_v)
            pltpu.sync_copy(sb_hbm.at[pl.ds(ab, HW)], sb_v)
            pltpu.async_copy(eo_hbm.at[row_v], rows_v, sem).wait()

            def body(i, carry):
                s0 = sb_v[2 * i, :]
                s1 = sb_v[2 * i + 1, :]
                for j in range(H // 32):
                    w0 = rows_v[2 * i, pl.ds(j * 16, 16)]
                    w1 = rows_v[2 * i + 1, pl.ds(j * 16, 16)]
                    lo0 = lax.bitcast_convert_type(w0 << 16, jnp.float32)
                    lo1 = lax.bitcast_convert_type(w1 << 16, jnp.float32)
                    hi0 = lax.bitcast_convert_type(
                        w0 & jnp.int32(-65536), jnp.float32)
                    hi1 = lax.bitcast_convert_type(
                        w1 & jnp.int32(-65536), jnp.float32)
                    out_v[i, pl.ds(j * 16, 16)] = lo0 * s0 + lo1 * s1
                    out_v[i, pl.ds((j + H // 32) * 16, 16)] = (
                        hi0 * s0 + hi1 * s1)
                return carry

            lax.fori_loop(0, HW // K, body, 0)
            pltpu.sync_copy(out_v, out_hbm.at[pl.ds(tb, HW // K)])

    return k(eo_flat, rowc, scale_b)


@jax.jit
def kernel(hidden_states, topk_weights, topk_ids, gate_up_weight, down_weight):
    ids_flat = topk_ids.reshape(A, 1).astype(jnp.int32)
    w_flat = topk_weights.reshape(A, 1)
    rowd, rowc, scale, tok, hb32 = _routing(ids_flat, w_flat, hidden_states)
    expert_in = _sc_dispatch(hb32, tok.reshape(A), rowd.reshape(A))
    eo = _ffn(expert_in.reshape(E + 1, C, H // 2), gate_up_weight, down_weight)
    return _sc_combine(eo.reshape(E * C, H // 2), rowc.reshape(A), scale)


# speedup vs baseline: 2.1307x; 1.0088x over previous
import functools

import jax
import jax.numpy as jnp
from jax import lax
from jax.experimental import pallas as pl
from jax.experimental.pallas import tpu as pltpu
from jax.experimental.pallas import tpu_sc as plsc

H = 768
F = 512
E = 64
K = 2
C = 192
T = 2048
A = T * K
CHUNK = 128
NCH = A // CHUNK


def _routing_body(ids_ref, w_ref, hid_ref, rowd_ref, rowc_ref, scale_ref,
                  tok_ref, hb_ref, oh_ref, cum_ref):
    xb = hid_ref[...].astype(jnp.bfloat16)
    lo = lax.bitcast_convert_type(xb[:, :H // 2], jnp.uint16).astype(jnp.uint32)
    hi = lax.bitcast_convert_type(xb[:, H // 2:], jnp.uint16).astype(jnp.uint32)
    hb_ref[...] = lax.bitcast_convert_type(lo | (hi << 16), jnp.int32)

    ids = ids_ref[...]
    eidx = lax.broadcasted_iota(jnp.int32, (1, E), 1)
    oh_ref[...] = (ids == eidx).astype(jnp.float32)
    tri = (lax.broadcasted_iota(jnp.int32, (CHUNK, CHUNK), 0)
           >= lax.broadcasted_iota(jnp.int32, (CHUNK, CHUNK), 1)
           ).astype(jnp.float32)

    def step(i, carry):
        oh_c = oh_ref[pl.ds(i * CHUNK, CHUNK), :]
        cum = lax.dot_general(tri, oh_c, (((1,), (0,)), ((), ())),
                              preferred_element_type=jnp.float32) + carry
        cum_ref[pl.ds(i * CHUNK, CHUNK), :] = cum
        return lax.slice(cum, (CHUNK - 1, 0), (CHUNK, E))

    lax.fori_loop(0, NCH, step, jnp.zeros((1, E), jnp.float32))

    pos = (jnp.sum(cum_ref[...] * oh_ref[...], axis=1, keepdims=True)
           .astype(jnp.int32) - 1)
    valid = pos < C
    slot = jnp.where(valid, pos, 0)
    rowc_ref[...] = ids * C + slot
    rowd_ref[...] = jnp.where(valid, ids * C + pos, E * C)
    scale = jnp.where(valid, w_ref[...], 0.0)
    scale_ref[...] = jnp.broadcast_to(scale, (A, 16))
    tok_ref[...] = lax.broadcasted_iota(jnp.int32, (A, 1), 0) // K


def _routing(ids_flat, w_flat, hidden):
    i32 = jnp.int32
    return pl.pallas_call(
        _routing_body,
        out_shape=[
            jax.ShapeDtypeStruct((A, 1), i32),
            jax.ShapeDtypeStruct((A, 1), i32),
            jax.ShapeDtypeStruct((A, 16), jnp.float32),
            jax.ShapeDtypeStruct((A, 1), i32),
            jax.ShapeDtypeStruct((T, H // 2), i32),
        ],
        scratch_shapes=[
            pltpu.VMEM((A, E), jnp.float32),
            pltpu.VMEM((A, E), jnp.float32),
        ],
    )(ids_flat, w_flat, hidden)


def _sc_dispatch(hidden, tok, rowd):
    info = plsc.get_sparse_core_info()
    nc = info.num_cores
    mesh = plsc.VectorSubcoreMesh(core_axis_name="c", subcore_axis_name="s")

    @functools.partial(
        pl.kernel, mesh=mesh,
        out_type=jax.ShapeDtypeStruct(((E + 1) * C, H // 2), jnp.int32),
        scratch_types=[
            pltpu.VMEM((CHUNK,), jnp.int32),
            pltpu.VMEM((CHUNK,), jnp.int32),
            pltpu.VMEM((CHUNK, H // 2), jnp.int32),
            pltpu.SemaphoreType.DMA,
        ],
    )
    def k(hid_hbm, tok_hbm, rowd_hbm, out_hbm, tok_v, row_v, rows_v, sem):
        wid = lax.axis_index("s") * nc + lax.axis_index("c")
        base = wid * CHUNK
        pltpu.sync_copy(tok_hbm.at[pl.ds(base, CHUNK)], tok_v)
        pltpu.sync_copy(rowd_hbm.at[pl.ds(base, CHUNK)], row_v)
        pltpu.async_copy(hid_hbm.at[tok_v], rows_v, sem).wait()
        pltpu.async_copy(rows_v, out_hbm.at[row_v], sem).wait()

    return k(hidden, tok, rowd)


def _ffn_body(x_ref, gw_ref, dw_ref, out_ref):
    xu = lax.bitcast_convert_type(x_ref[0], jnp.uint32)
    x_lo = lax.bitcast_convert_type((xu & 0xffff).astype(jnp.uint16),
                                    jnp.bfloat16)
    x_hi = lax.bitcast_convert_type((xu >> 16).astype(jnp.uint16),
                                    jnp.bfloat16)
    x = jnp.concatenate([x_lo, x_hi], axis=1)
    gw = gw_ref[0].astype(jnp.bfloat16)
    gu = lax.dot_general(x, gw, (((1,), (1,)), ((), ())),
                         preferred_element_type=jnp.float32)
    gate = gu[:, :F]
    up = gu[:, F:]
    act = (gate * jax.nn.sigmoid(gate) * up).astype(jnp.bfloat16)
    dw = dw_ref[0].astype(jnp.bfloat16)
    eo = lax.dot_general(act, dw, (((1,), (1,)), ((), ())),
                         preferred_element_type=jnp.float32)
    eb = eo.astype(jnp.bfloat16)
    lo = lax.bitcast_convert_type(eb[:, :H // 2], jnp.uint16).astype(jnp.uint32)
    hi = lax.bitcast_convert_type(eb[:, H // 2:], jnp.uint16).astype(jnp.uint32)
    out_ref[0] = lax.bitcast_convert_type(lo | (hi << 16), jnp.int32)


def _ffn(expert_in, gate_up_weight, down_weight):
    return pl.pallas_call(
        _ffn_body,
        grid=(E,),
        in_specs=[
            pl.BlockSpec((1, C, H // 2), lambda e: (e, 0, 0)),
            pl.BlockSpec((1, 2 * F, H), lambda e: (e, 0, 0)),
            pl.BlockSpec((1, H, F), lambda e: (e, 0, 0)),
        ],
        out_specs=pl.BlockSpec((1, C, H // 2), lambda e: (e, 0, 0)),
        out_shape=jax.ShapeDtypeStruct((E, C, H // 2), jnp.int32),
    )(expert_in, gate_up_weight, down_weight)


HW = 64


def _sc_combine(eo_flat, rowc, scale_b):
    info = plsc.get_sparse_core_info()
    nc = info.num_cores
    mesh = plsc.VectorSubcoreMesh(core_axis_name="c", subcore_axis_name="s")

    @functools.partial(
        pl.kernel, mesh=mesh,
        out_type=jax.ShapeDtypeStruct((T, H), jnp.float32),
        scratch_types=[
            pltpu.VMEM((HW,), jnp.int32),
            pltpu.VMEM((HW,), jnp.int32),
            pltpu.VMEM((CHUNK, 16), jnp.float32),
            pltpu.VMEM((HW, H // 2), jnp.int32),
            pltpu.VMEM((HW, H // 2), jnp.int32),
            pltpu.VMEM((HW // K, H), jnp.float32),
            pltpu.SemaphoreType.DMA,
            pltpu.SemaphoreType.DMA,
        ],
    )
    def k(eo_hbm, rowc_hbm, sb_hbm, out_hbm,
          row0_v, row1_v, sb_v, rows0_v, rows1_v, out_v, sem0, sem1):
        wid = lax.axis_index("s") * nc + lax.axis_index("c")
        base = pl.multiple_of(wid * CHUNK, CHUNK)
        pltpu.sync_copy(rowc_hbm.at[pl.ds(base, HW)], row0_v)
        pltpu.sync_copy(rowc_hbm.at[pl.ds(base + HW, HW)], row1_v)
        pltpu.sync_copy(sb_hbm.at[pl.ds(base, CHUNK)], sb_v)
        c0 = pltpu.async_copy(eo_hbm.at[row0_v], rows0_v, sem0)
        c1 = pltpu.async_copy(eo_hbm.at[row1_v], rows1_v, sem1)

        for h, (rows_v, cdma) in enumerate(((rows0_v, c0), (rows1_v, c1))):
            tb = pl.multiple_of((wid * CHUNK + h * HW) // K, HW // K)
            cdma.wait()

            def body(i, carry):
                s0 = sb_v[h * HW + 2 * i, :]
                s1 = sb_v[h * HW + 2 * i + 1, :]
                for j in range(H // 32):
                    w0 = rows_v[2 * i, pl.ds(j * 16, 16)]
                    w1 = rows_v[2 * i + 1, pl.ds(j * 16, 16)]
                    lo0 = lax.bitcast_convert_type(w0 << 16, jnp.float32)
                    lo1 = lax.bitcast_convert_type(w1 << 16, jnp.float32)
                    hi0 = lax.bitcast_convert_type(
                        w0 & jnp.int32(-65536), jnp.float32)
                    hi1 = lax.bitcast_convert_type(
                        w1 & jnp.int32(-65536), jnp.float32)
                    out_v[i, pl.ds(j * 16, 16)] = lo0 * s0 + lo1 * s1
                    out_v[i, pl.ds((j + H // 32) * 16, 16)] = (
                        hi0 * s0 + hi1 * s1)
                return carry

            lax.fori_loop(0, HW // K, body, 0)
            pltpu.sync_copy(out_v, out_hbm.at[pl.ds(tb, HW // K)])

    return k(eo_flat, rowc, scale_b)


@jax.jit
def kernel(hidden_states, topk_weights, topk_ids, gate_up_weight, down_weight):
    ids_flat = topk_ids.reshape(A, 1).astype(jnp.int32)
    w_flat = topk_weights.reshape(A, 1)
    rowd, rowc, scale, tok, hb32 = _routing(ids_flat, w_flat, hidden_states)
    expert_in = _sc_dispatch(hb32, tok.reshape(A), rowd.reshape(A))
    eo = _ffn(expert_in.reshape(E + 1, C, H // 2), gate_up_weight, down_weight)
    return _sc_combine(eo.reshape(E * C, H // 2), rowc.reshape(A), scale)


# speedup vs baseline: 2.1967x; 1.0310x over previous
import functools

import jax
import jax.numpy as jnp
from jax import lax
from jax.experimental import pallas as pl
from jax.experimental.pallas import tpu as pltpu
from jax.experimental.pallas import tpu_sc as plsc

H = 768
F = 512
E = 64
K = 2
C = 192
T = 2048
A = T * K
CHUNK = 128
NCH = A // CHUNK


def _routing_body(ids_ref, w_ref, hid_ref, rowd_ref, rowc_ref, scale_ref,
                  tok_ref, hb_ref, oh_ref, cum_ref):
    xb = hid_ref[...].astype(jnp.bfloat16)
    lo = lax.bitcast_convert_type(xb[:, :H // 2], jnp.uint16).astype(jnp.uint32)
    hi = lax.bitcast_convert_type(xb[:, H // 2:], jnp.uint16).astype(jnp.uint32)
    hb_ref[...] = lax.bitcast_convert_type(lo | (hi << 16), jnp.int32)

    ids = ids_ref[...]
    eidx = lax.broadcasted_iota(jnp.int32, (1, E), 1)
    oh_ref[...] = (ids == eidx).astype(jnp.float32)
    tri = (lax.broadcasted_iota(jnp.int32, (CHUNK, CHUNK), 0)
           >= lax.broadcasted_iota(jnp.int32, (CHUNK, CHUNK), 1)
           ).astype(jnp.float32)

    def step(i, carry):
        oh_c = oh_ref[pl.ds(i * CHUNK, CHUNK), :]
        cum = lax.dot_general(tri, oh_c, (((1,), (0,)), ((), ())),
                              preferred_element_type=jnp.float32) + carry
        cum_ref[pl.ds(i * CHUNK, CHUNK), :] = cum
        return lax.slice(cum, (CHUNK - 1, 0), (CHUNK, E))

    lax.fori_loop(0, NCH, step, jnp.zeros((1, E), jnp.float32))

    pos = (jnp.sum(cum_ref[...] * oh_ref[...], axis=1, keepdims=True)
           .astype(jnp.int32) - 1)
    valid = pos < C
    slot = jnp.where(valid, pos, 0)
    rowc_ref[...] = ids * C + slot
    rowd_ref[...] = jnp.where(valid, ids * C + pos, E * C)
    scale = jnp.where(valid, w_ref[...], 0.0)
    scale_ref[...] = jnp.broadcast_to(scale, (A, 16))
    tok_ref[...] = lax.broadcasted_iota(jnp.int32, (A, 1), 0) // K


def _routing(ids_flat, w_flat, hidden):
    i32 = jnp.int32
    return pl.pallas_call(
        _routing_body,
        out_shape=[
            jax.ShapeDtypeStruct((A, 1), i32),
            jax.ShapeDtypeStruct((A, 1), i32),
            jax.ShapeDtypeStruct((A, 16), jnp.float32),
            jax.ShapeDtypeStruct((A, 1), i32),
            jax.ShapeDtypeStruct((T, H // 2), i32),
        ],
        scratch_shapes=[
            pltpu.VMEM((A, E), jnp.float32),
            pltpu.VMEM((A, E), jnp.float32),
        ],
    )(ids_flat, w_flat, hidden)


def _sc_dispatch(hidden, tok, rowd):
    info = plsc.get_sparse_core_info()
    nc = info.num_cores
    mesh = plsc.VectorSubcoreMesh(core_axis_name="c", subcore_axis_name="s")

    @functools.partial(
        pl.kernel, mesh=mesh,
        out_type=jax.ShapeDtypeStruct(((E + 1) * C, H // 2), jnp.int32),
        scratch_types=[
            pltpu.VMEM((CHUNK,), jnp.int32),
            pltpu.VMEM((CHUNK,), jnp.int32),
            pltpu.VMEM((CHUNK, H // 2), jnp.int32),
            pltpu.SemaphoreType.DMA,
        ],
    )
    def k(hid_hbm, tok_hbm, rowd_hbm, out_hbm, tok_v, row_v, rows_v, sem):
        wid = lax.axis_index("s") * nc + lax.axis_index("c")
        base = wid * CHUNK
        pltpu.sync_copy(tok_hbm.at[pl.ds(base, CHUNK)], tok_v)
        pltpu.sync_copy(rowd_hbm.at[pl.ds(base, CHUNK)], row_v)
        pltpu.async_copy(hid_hbm.at[tok_v], rows_v, sem).wait()
        pltpu.async_copy(rows_v, out_hbm.at[row_v], sem).wait()

    return k(hidden, tok, rowd)


def _ffn_body(x_ref, gw_ref, dw_ref, out_ref):
    xu = lax.bitcast_convert_type(x_ref[0], jnp.uint32)
    x_lo = lax.bitcast_convert_type((xu & 0xffff).astype(jnp.uint16),
                                    jnp.bfloat16)
    x_hi = lax.bitcast_convert_type((xu >> 16).astype(jnp.uint16),
                                    jnp.bfloat16)
    x = jnp.concatenate([x_lo, x_hi], axis=1)
    gw = gw_ref[0].astype(jnp.bfloat16)
    gu = lax.dot_general(x, gw, (((1,), (1,)), ((), ())),
                         preferred_element_type=jnp.float32)
    gate = gu[:, :F]
    up = gu[:, F:]
    act = (gate * jax.nn.sigmoid(gate) * up).astype(jnp.bfloat16)
    dw = dw_ref[0].astype(jnp.bfloat16)
    eo = lax.dot_general(act, dw, (((1,), (1,)), ((), ())),
                         preferred_element_type=jnp.float32)
    eb = eo.astype(jnp.bfloat16)
    lo = lax.bitcast_convert_type(eb[:, :H // 2], jnp.uint16).astype(jnp.uint32)
    hi = lax.bitcast_convert_type(eb[:, H // 2:], jnp.uint16).astype(jnp.uint32)
    out_ref[0] = lax.bitcast_convert_type(lo | (hi << 16), jnp.int32)


def _ffn(expert_in, gate_up_weight, down_weight):
    return pl.pallas_call(
        _ffn_body,
        grid=(E,),
        in_specs=[
            pl.BlockSpec((1, C, H // 2), lambda e: (e, 0, 0)),
            pl.BlockSpec((1, 2 * F, H), lambda e: (e, 0, 0)),
            pl.BlockSpec((1, H, F), lambda e: (e, 0, 0)),
        ],
        out_specs=pl.BlockSpec((1, C, H // 2), lambda e: (e, 0, 0)),
        out_shape=jax.ShapeDtypeStruct((E, C, H // 2), jnp.int32),
    )(expert_in, gate_up_weight, down_weight)


HW = 64


def _sc_combine(eo_flat, rowc, scale_b):
    info = plsc.get_sparse_core_info()
    nc = info.num_cores
    mesh = plsc.VectorSubcoreMesh(core_axis_name="c", subcore_axis_name="s")

    @functools.partial(
        pl.kernel, mesh=mesh,
        out_type=jax.ShapeDtypeStruct((T, H), jnp.float32),
        scratch_types=[
            pltpu.VMEM((HW,), jnp.int32),
            pltpu.VMEM((HW,), jnp.int32),
            pltpu.VMEM((CHUNK, 16), jnp.float32),
            pltpu.VMEM((HW, H // 2), jnp.int32),
            pltpu.VMEM((HW, H // 2), jnp.int32),
            pltpu.VMEM((HW // K, H), jnp.float32),
            pltpu.SemaphoreType.DMA,
            pltpu.SemaphoreType.DMA,
        ],
    )
    def k(eo_hbm, rowc_hbm, sb_hbm, out_hbm,
          row0_v, row1_v, sb_v, rows0_v, rows1_v, out_v, sem0, sem1):
        wid = lax.axis_index("s") * nc + lax.axis_index("c")
        base = pl.multiple_of(wid * CHUNK, CHUNK)
        pltpu.sync_copy(rowc_hbm.at[pl.ds(base, HW)], row0_v)
        pltpu.sync_copy(rowc_hbm.at[pl.ds(base + HW, HW)], row1_v)
        pltpu.sync_copy(sb_hbm.at[pl.ds(base, CHUNK)], sb_v)
        c0 = pltpu.async_copy(eo_hbm.at[row0_v], rows0_v, sem0)
        c1 = pltpu.async_copy(eo_hbm.at[row1_v], rows1_v, sem1)

        for h, (rows_v, cdma) in enumerate(((rows0_v, c0), (rows1_v, c1))):
            tb = pl.multiple_of((wid * CHUNK + h * HW) // K, HW // K)
            cdma.wait()

            @plsc.parallel_loop(0, HW // K, unroll=2)
            def body(i):
                s0 = sb_v[h * HW + 2 * i, :]
                s1 = sb_v[h * HW + 2 * i + 1, :]
                for j in range(H // 32):
                    w0 = rows_v[2 * i, pl.ds(j * 16, 16)]
                    w1 = rows_v[2 * i + 1, pl.ds(j * 16, 16)]
                    lo0 = lax.bitcast_convert_type(w0 << 16, jnp.float32)
                    lo1 = lax.bitcast_convert_type(w1 << 16, jnp.float32)
                    hi0 = lax.bitcast_convert_type(
                        w0 & jnp.int32(-65536), jnp.float32)
                    hi1 = lax.bitcast_convert_type(
                        w1 & jnp.int32(-65536), jnp.float32)
                    out_v[i, pl.ds(j * 16, 16)] = lo0 * s0 + lo1 * s1
                    out_v[i, pl.ds((j + H // 32) * 16, 16)] = (
                        hi0 * s0 + hi1 * s1)
            pltpu.sync_copy(out_v, out_hbm.at[pl.ds(tb, HW // K)])

    return k(eo_flat, rowc, scale_b)


@jax.jit
def kernel(hidden_states, topk_weights, topk_ids, gate_up_weight, down_weight):
    ids_flat = topk_ids.reshape(A, 1).astype(jnp.int32)
    w_flat = topk_weights.reshape(A, 1)
    rowd, rowc, scale, tok, hb32 = _routing(ids_flat, w_flat, hidden_states)
    expert_in = _sc_dispatch(hb32, tok.reshape(A), rowd.reshape(A))
    eo = _ffn(expert_in.reshape(E + 1, C, H // 2), gate_up_weight, down_weight)
    return _sc_combine(eo.reshape(E * C, H // 2), rowc.reshape(A), scale)


# speedup vs baseline: 2.4310x; 1.1067x over previous
import functools

import jax
import jax.numpy as jnp
from jax import lax
from jax.experimental import pallas as pl
from jax.experimental.pallas import tpu as pltpu
from jax.experimental.pallas import tpu_sc as plsc

H = 768
F = 512
E = 64
K = 2
C = 192
T = 2048
A = T * K
CHUNK = 128
NCH = A // CHUNK


def _routing_body(ids_ref, w_ref, hid_ref, rowd_ref, rowc_ref, scale_ref,
                  tok_ref, hb_ref, oh_ref, cum_ref):
    xb = hid_ref[...].astype(jnp.bfloat16)
    lo = lax.bitcast_convert_type(xb[:, :H // 2], jnp.uint16).astype(jnp.uint32)
    hi = lax.bitcast_convert_type(xb[:, H // 2:], jnp.uint16).astype(jnp.uint32)
    hb_ref[...] = lax.bitcast_convert_type(lo | (hi << 16), jnp.int32)

    ids = ids_ref[...]
    eidx = lax.broadcasted_iota(jnp.int32, (1, E), 1)
    oh_ref[...] = (ids == eidx).astype(jnp.float32)
    tri = (lax.broadcasted_iota(jnp.int32, (CHUNK, CHUNK), 0)
           >= lax.broadcasted_iota(jnp.int32, (CHUNK, CHUNK), 1)
           ).astype(jnp.float32)

    def step(i, carry):
        oh_c = oh_ref[pl.ds(i * CHUNK, CHUNK), :]
        cum = lax.dot_general(tri, oh_c, (((1,), (0,)), ((), ())),
                              preferred_element_type=jnp.float32) + carry
        cum_ref[pl.ds(i * CHUNK, CHUNK), :] = cum
        return lax.slice(cum, (CHUNK - 1, 0), (CHUNK, E))

    lax.fori_loop(0, NCH, step, jnp.zeros((1, E), jnp.float32))

    pos = (jnp.sum(cum_ref[...] * oh_ref[...], axis=1, keepdims=True)
           .astype(jnp.int32) - 1)
    valid = pos < C
    slot = jnp.where(valid, pos, 0)
    rowc_ref[...] = ids * C + slot
    rowd_ref[...] = jnp.where(valid, ids * C + pos, E * C)
    scale = jnp.where(valid, w_ref[...], 0.0)
    scale_ref[...] = jnp.broadcast_to(scale, (A, 16))
    tok_ref[...] = lax.broadcasted_iota(jnp.int32, (A, 1), 0) // K


def _routing(ids_flat, w_flat, hidden):
    i32 = jnp.int32
    return pl.pallas_call(
        _routing_body,
        out_shape=[
            jax.ShapeDtypeStruct((A, 1), i32),
            jax.ShapeDtypeStruct((A, 1), i32),
            jax.ShapeDtypeStruct((A, 16), jnp.float32),
            jax.ShapeDtypeStruct((A, 1), i32),
            jax.ShapeDtypeStruct((T, H // 2), i32),
        ],
        scratch_shapes=[
            pltpu.VMEM((A, E), jnp.float32),
            pltpu.VMEM((A, E), jnp.float32),
        ],
    )(ids_flat, w_flat, hidden)


def _sc_dispatch(hidden, tok, rowd):
    info = plsc.get_sparse_core_info()
    nc = info.num_cores
    mesh = plsc.VectorSubcoreMesh(core_axis_name="c", subcore_axis_name="s")

    @functools.partial(
        pl.kernel, mesh=mesh,
        out_type=jax.ShapeDtypeStruct(((E + 1) * C, H // 2), jnp.int32),
        scratch_types=[
            pltpu.VMEM((CHUNK,), jnp.int32),
            pltpu.VMEM((CHUNK,), jnp.int32),
            pltpu.VMEM((CHUNK, H // 2), jnp.int32),
            pltpu.SemaphoreType.DMA,
        ],
    )
    def k(hid_hbm, tok_hbm, rowd_hbm, out_hbm, tok_v, row_v, rows_v, sem):
        wid = lax.axis_index("s") * nc + lax.axis_index("c")
        base = wid * CHUNK
        pltpu.sync_copy(tok_hbm.at[pl.ds(base, CHUNK)], tok_v)
        pltpu.sync_copy(rowd_hbm.at[pl.ds(base, CHUNK)], row_v)
        pltpu.async_copy(hid_hbm.at[tok_v], rows_v, sem).wait()
        pltpu.async_copy(rows_v, out_hbm.at[row_v], sem).wait()

    return k(hidden, tok, rowd)


EB = 2


def _ffn_body(x_ref, gw_ref, dw_ref, out_ref):
    for e in range(EB):
        xu = lax.bitcast_convert_type(x_ref[e], jnp.uint32)
        x_lo = lax.bitcast_convert_type((xu & 0xffff).astype(jnp.uint16),
                                        jnp.bfloat16)
        x_hi = lax.bitcast_convert_type((xu >> 16).astype(jnp.uint16),
                                        jnp.bfloat16)
        x = jnp.concatenate([x_lo, x_hi], axis=1)
        gw = gw_ref[e].astype(jnp.bfloat16)
        gu = lax.dot_general(x, gw, (((1,), (1,)), ((), ())),
                             preferred_element_type=jnp.float32)
        gate = gu[:, :F]
        up = gu[:, F:]
        act = (gate * jax.nn.sigmoid(gate) * up).astype(jnp.bfloat16)
        dw = dw_ref[e].astype(jnp.bfloat16)
        eo = lax.dot_general(act, dw, (((1,), (1,)), ((), ())),
                             preferred_element_type=jnp.float32)
        eb16 = eo.astype(jnp.bfloat16)
        lo = lax.bitcast_convert_type(eb16[:, :H // 2],
                                      jnp.uint16).astype(jnp.uint32)
        hi = lax.bitcast_convert_type(eb16[:, H // 2:],
                                      jnp.uint16).astype(jnp.uint32)
        out_ref[e] = lax.bitcast_convert_type(lo | (hi << 16), jnp.int32)


def _ffn(expert_in, gate_up_weight, down_weight):
    return pl.pallas_call(
        _ffn_body,
        grid=(E // EB,),
        in_specs=[
            pl.BlockSpec((EB, C, H // 2), lambda e: (e, 0, 0)),
            pl.BlockSpec((EB, 2 * F, H), lambda e: (e, 0, 0)),
            pl.BlockSpec((EB, H, F), lambda e: (e, 0, 0)),
        ],
        out_specs=pl.BlockSpec((EB, C, H // 2), lambda e: (e, 0, 0)),
        out_shape=jax.ShapeDtypeStruct((E, C, H // 2), jnp.int32),
    )(expert_in, gate_up_weight, down_weight)


HW = 64


def _sc_combine(eo_flat, rowc, scale_b):
    info = plsc.get_sparse_core_info()
    nc = info.num_cores
    mesh = plsc.VectorSubcoreMesh(core_axis_name="c", subcore_axis_name="s")

    @functools.partial(
        pl.kernel, mesh=mesh,
        out_type=jax.ShapeDtypeStruct((T, H), jnp.float32),
        scratch_types=[
            pltpu.VMEM((HW,), jnp.int32),
            pltpu.VMEM((HW,), jnp.int32),
            pltpu.VMEM((CHUNK, 16), jnp.float32),
            pltpu.VMEM((HW, H // 2), jnp.int32),
            pltpu.VMEM((HW, H // 2), jnp.int32),
            pltpu.VMEM((HW // K, H), jnp.float32),
            pltpu.SemaphoreType.DMA,
            pltpu.SemaphoreType.DMA,
        ],
    )
    def k(eo_hbm, rowc_hbm, sb_hbm, out_hbm,
          row0_v, row1_v, sb_v, rows0_v, rows1_v, out_v, sem0, sem1):
        wid = lax.axis_index("s") * nc + lax.axis_index("c")
        base = pl.multiple_of(wid * CHUNK, CHUNK)
        pltpu.sync_copy(rowc_hbm.at[pl.ds(base, HW)], row0_v)
        pltpu.sync_copy(rowc_hbm.at[pl.ds(base + HW, HW)], row1_v)
        pltpu.sync_copy(sb_hbm.at[pl.ds(base, CHUNK)], sb_v)
        c0 = pltpu.async_copy(eo_hbm.at[row0_v], rows0_v, sem0)
        c1 = pltpu.async_copy(eo_hbm.at[row1_v], rows1_v, sem1)

        for h, (rows_v, cdma) in enumerate(((rows0_v, c0), (rows1_v, c1))):
            tb = pl.multiple_of((wid * CHUNK + h * HW) // K, HW // K)
            cdma.wait()

            @plsc.parallel_loop(0, HW // K, unroll=2)
            def body(i):
                s0 = sb_v[h * HW + 2 * i, :]
                s1 = sb_v[h * HW + 2 * i + 1, :]
                for j in range(H // 32):
                    w0 = rows_v[2 * i, pl.ds(j * 16, 16)]
                    w1 = rows_v[2 * i + 1, pl.ds(j * 16, 16)]
                    lo0 = lax.bitcast_convert_type(w0 << 16, jnp.float32)
                    lo1 = lax.bitcast_convert_type(w1 << 16, jnp.float32)
                    hi0 = lax.bitcast_convert_type(
                        w0 & jnp.int32(-65536), jnp.float32)
                    hi1 = lax.bitcast_convert_type(
                        w1 & jnp.int32(-65536), jnp.float32)
                    out_v[i, pl.ds(j * 16, 16)] = lo0 * s0 + lo1 * s1
                    out_v[i, pl.ds((j + H // 32) * 16, 16)] = (
                        hi0 * s0 + hi1 * s1)
            pltpu.sync_copy(out_v, out_hbm.at[pl.ds(tb, HW // K)])

    return k(eo_flat, rowc, scale_b)


@jax.jit
def kernel(hidden_states, topk_weights, topk_ids, gate_up_weight, down_weight):
    ids_flat = topk_ids.reshape(A, 1).astype(jnp.int32)
    w_flat = topk_weights.reshape(A, 1)
    rowd, rowc, scale, tok, hb32 = _routing(ids_flat, w_flat, hidden_states)
    expert_in = _sc_dispatch(hb32, tok.reshape(A), rowd.reshape(A))
    eo = _ffn(expert_in.reshape(E + 1, C, H // 2), gate_up_weight, down_weight)
    return _sc_combine(eo.reshape(E * C, H // 2), rowc.reshape(A), scale)


# speedup vs baseline: 2.4432x; 1.0050x over previous
import functools

import jax
import jax.numpy as jnp
from jax import lax
from jax.experimental import pallas as pl
from jax.experimental.pallas import tpu as pltpu
from jax.experimental.pallas import tpu_sc as plsc

H = 768
F = 512
E = 64
K = 2
C = 192
T = 2048
A = T * K
CHUNK = 128
NCH = A // CHUNK


def _routing_body(ids_ref, w_ref, hid_ref, rowd_ref, rowc_ref, scale_ref,
                  tok_ref, hb_ref, oh_ref, cum_ref):
    xb = hid_ref[...].astype(jnp.bfloat16)
    lo = lax.bitcast_convert_type(xb[:, :H // 2], jnp.uint16).astype(jnp.uint32)
    hi = lax.bitcast_convert_type(xb[:, H // 2:], jnp.uint16).astype(jnp.uint32)
    hb_ref[...] = lax.bitcast_convert_type(lo | (hi << 16), jnp.int32)

    ids = ids_ref[...]
    eidx = lax.broadcasted_iota(jnp.int32, (1, E), 1)
    oh_ref[...] = (ids == eidx).astype(jnp.float32)
    tri = (lax.broadcasted_iota(jnp.int32, (CHUNK, CHUNK), 0)
           >= lax.broadcasted_iota(jnp.int32, (CHUNK, CHUNK), 1)
           ).astype(jnp.float32)

    def step(i, carry):
        oh_c = oh_ref[pl.ds(i * CHUNK, CHUNK), :]
        cum = lax.dot_general(tri, oh_c, (((1,), (0,)), ((), ())),
                              preferred_element_type=jnp.float32) + carry
        cum_ref[pl.ds(i * CHUNK, CHUNK), :] = cum
        return lax.slice(cum, (CHUNK - 1, 0), (CHUNK, E))

    lax.fori_loop(0, NCH, step, jnp.zeros((1, E), jnp.float32))

    pos = (jnp.sum(cum_ref[...] * oh_ref[...], axis=1, keepdims=True)
           .astype(jnp.int32) - 1)
    valid = pos < C
    slot = jnp.where(valid, pos, 0)
    rowc_ref[...] = ids * C + slot
    rowd_ref[...] = jnp.where(valid, ids * C + pos, E * C)
    scale = jnp.where(valid, w_ref[...], 0.0)
    scale_ref[...] = jnp.broadcast_to(scale, (A, 16))
    tok_ref[...] = lax.broadcasted_iota(jnp.int32, (A, 1), 0) // K


def _routing(ids_flat, w_flat, hidden):
    i32 = jnp.int32
    return pl.pallas_call(
        _routing_body,
        out_shape=[
            jax.ShapeDtypeStruct((A, 1), i32),
            jax.ShapeDtypeStruct((A, 1), i32),
            jax.ShapeDtypeStruct((A, 16), jnp.float32),
            jax.ShapeDtypeStruct((A, 1), i32),
            jax.ShapeDtypeStruct((T, H // 2), i32),
        ],
        scratch_shapes=[
            pltpu.VMEM((A, E), jnp.float32),
            pltpu.VMEM((A, E), jnp.float32),
        ],
    )(ids_flat, w_flat, hidden)


def _sc_dispatch(hidden, tok, rowd):
    info = plsc.get_sparse_core_info()
    nc = info.num_cores
    mesh = plsc.VectorSubcoreMesh(core_axis_name="c", subcore_axis_name="s")

    @functools.partial(
        pl.kernel, mesh=mesh,
        out_type=jax.ShapeDtypeStruct(((E + 1) * C, H // 2), jnp.int32),
        scratch_types=[
            pltpu.VMEM((CHUNK,), jnp.int32),
            pltpu.VMEM((CHUNK,), jnp.int32),
            pltpu.VMEM((CHUNK, H // 2), jnp.int32),
            pltpu.SemaphoreType.DMA,
        ],
    )
    def k(hid_hbm, tok_hbm, rowd_hbm, out_hbm, tok_v, row_v, rows_v, sem):
        wid = lax.axis_index("s") * nc + lax.axis_index("c")
        base = wid * CHUNK
        pltpu.sync_copy(tok_hbm.at[pl.ds(base, CHUNK)], tok_v)
        pltpu.sync_copy(rowd_hbm.at[pl.ds(base, CHUNK)], row_v)
        pltpu.async_copy(hid_hbm.at[tok_v], rows_v, sem).wait()
        pltpu.async_copy(rows_v, out_hbm.at[row_v], sem).wait()

    return k(hidden, tok, rowd)


EB = 4


def _ffn_body(x_ref, gw_ref, dw_ref, out_ref):
    for e in range(EB):
        xu = lax.bitcast_convert_type(x_ref[e], jnp.uint32)
        x_lo = lax.bitcast_convert_type((xu & 0xffff).astype(jnp.uint16),
                                        jnp.bfloat16)
        x_hi = lax.bitcast_convert_type((xu >> 16).astype(jnp.uint16),
                                        jnp.bfloat16)
        x = jnp.concatenate([x_lo, x_hi], axis=1)
        gw = gw_ref[e].astype(jnp.bfloat16)
        gu = lax.dot_general(x, gw, (((1,), (1,)), ((), ())),
                             preferred_element_type=jnp.float32)
        gate = gu[:, :F]
        up = gu[:, F:]
        act = (gate * jax.nn.sigmoid(gate) * up).astype(jnp.bfloat16)
        dw = dw_ref[e].astype(jnp.bfloat16)
        eo = lax.dot_general(act, dw, (((1,), (1,)), ((), ())),
                             preferred_element_type=jnp.float32)
        eb16 = eo.astype(jnp.bfloat16)
        lo = lax.bitcast_convert_type(eb16[:, :H // 2],
                                      jnp.uint16).astype(jnp.uint32)
        hi = lax.bitcast_convert_type(eb16[:, H // 2:],
                                      jnp.uint16).astype(jnp.uint32)
        out_ref[e] = lax.bitcast_convert_type(lo | (hi << 16), jnp.int32)


def _ffn(expert_in, gate_up_weight, down_weight):
    return pl.pallas_call(
        _ffn_body,
        grid=(E // EB,),
        in_specs=[
            pl.BlockSpec((EB, C, H // 2), lambda e: (e, 0, 0)),
            pl.BlockSpec((EB, 2 * F, H), lambda e: (e, 0, 0)),
            pl.BlockSpec((EB, H, F), lambda e: (e, 0, 0)),
        ],
        out_specs=pl.BlockSpec((EB, C, H // 2), lambda e: (e, 0, 0)),
        out_shape=jax.ShapeDtypeStruct((E, C, H // 2), jnp.int32),
    )(expert_in, gate_up_weight, down_weight)


HW = 64


def _sc_combine(eo_flat, rowc, scale_b):
    info = plsc.get_sparse_core_info()
    nc = info.num_cores
    mesh = plsc.VectorSubcoreMesh(core_axis_name="c", subcore_axis_name="s")

    @functools.partial(
        pl.kernel, mesh=mesh,
        out_type=jax.ShapeDtypeStruct((T, H), jnp.float32),
        scratch_types=[
            pltpu.VMEM((HW,), jnp.int32),
            pltpu.VMEM((HW,), jnp.int32),
            pltpu.VMEM((CHUNK, 16), jnp.float32),
            pltpu.VMEM((HW, H // 2), jnp.int32),
            pltpu.VMEM((HW, H // 2), jnp.int32),
            pltpu.VMEM((HW // K, H), jnp.float32),
            pltpu.SemaphoreType.DMA,
            pltpu.SemaphoreType.DMA,
        ],
    )
    def k(eo_hbm, rowc_hbm, sb_hbm, out_hbm,
          row0_v, row1_v, sb_v, rows0_v, rows1_v, out_v, sem0, sem1):
        wid = lax.axis_index("s") * nc + lax.axis_index("c")
        base = pl.multiple_of(wid * CHUNK, CHUNK)
        pltpu.sync_copy(rowc_hbm.at[pl.ds(base, HW)], row0_v)
        pltpu.sync_copy(rowc_hbm.at[pl.ds(base + HW, HW)], row1_v)
        pltpu.sync_copy(sb_hbm.at[pl.ds(base, CHUNK)], sb_v)
        c0 = pltpu.async_copy(eo_hbm.at[row0_v], rows0_v, sem0)
        c1 = pltpu.async_copy(eo_hbm.at[row1_v], rows1_v, sem1)

        for h, (rows_v, cdma) in enumerate(((rows0_v, c0), (rows1_v, c1))):
            tb = pl.multiple_of((wid * CHUNK + h * HW) // K, HW // K)
            cdma.wait()

            @plsc.parallel_loop(0, HW // K, unroll=2)
            def body(i):
                s0 = sb_v[h * HW + 2 * i, :]
                s1 = sb_v[h * HW + 2 * i + 1, :]
                for j in range(H // 32):
                    w0 = rows_v[2 * i, pl.ds(j * 16, 16)]
                    w1 = rows_v[2 * i + 1, pl.ds(j * 16, 16)]
                    lo0 = lax.bitcast_convert_type(w0 << 16, jnp.float32)
                    lo1 = lax.bitcast_convert_type(w1 << 16, jnp.float32)
                    hi0 = lax.bitcast_convert_type(
                        w0 & jnp.int32(-65536), jnp.float32)
                    hi1 = lax.bitcast_convert_type(
                        w1 & jnp.int32(-65536), jnp.float32)
                    out_v[i, pl.ds(j * 16, 16)] = lo0 * s0 + lo1 * s1
                    out_v[i, pl.ds((j + H // 32) * 16, 16)] = (
                        hi0 * s0 + hi1 * s1)
            pltpu.sync_copy(out_v, out_hbm.at[pl.ds(tb, HW // K)])

    return k(eo_flat, rowc, scale_b)


@jax.jit
def kernel(hidden_states, topk_weights, topk_ids, gate_up_weight, down_weight):
    ids_flat = topk_ids.reshape(A, 1).astype(jnp.int32)
    w_flat = topk_weights.reshape(A, 1)
    rowd, rowc, scale, tok, hb32 = _routing(ids_flat, w_flat, hidden_states)
    expert_in = _sc_dispatch(hb32, tok.reshape(A), rowd.reshape(A))
    eo = _ffn(expert_in.reshape(E + 1, C, H // 2), gate_up_weight, down_weight)
    return _sc_combine(eo.reshape(E * C, H // 2), rowc.reshape(A), scale)
